# Initial kernel scaffold; baseline (speedup 1.0000x reference)
import jax
import jax.numpy as jnp
from jax.experimental import pallas as pl


def kernel(xyz, new_xyz, batch_distances, inds, features):
    raise NotImplementedError("write your pallas kernel here")


# speedup vs baseline: 4.5956x; 4.5956x over previous
import dataclasses
import functools

import jax
import jax.numpy as jnp
from jax import lax
from jax.experimental import pallas as pl
from jax.experimental.pallas import tpu as pltpu
from jax.experimental.pallas import tpu_sc as plsc

_RADIUS2 = 0.1 * 0.1
_S = 32

_SC_CORES = 2
_SC_SUBCORES = 16
_SC_LANES = 16
_NW = _SC_CORES * _SC_SUBCORES


def _ball_body(xyzt_ref, q_ref, idx_ref, flat_ref, *, N, Mt):
    b = pl.program_id(0)
    px = xyzt_ref[0, 0:1, :]
    py = xyzt_ref[0, 1:2, :]
    pz = xyzt_ref[0, 2:3, :]
    q = q_ref[0]
    qx = q[:, 0:1]
    qy = q[:, 1:2]
    qz = q[:, 2:3]
    p2 = (px * px + py * py) + pz * pz
    q2 = (qx * qx + qy * qy) + qz * qz
    cross = jnp.dot(q, xyzt_ref[0], preferred_element_type=jnp.float32)
    dist2 = (q2 + p2) - 2.0 * cross
    maskf = (dist2 < _RADIUS2).astype(jnp.float32)

    c = maskf
    sh = 1
    while sh < N:
        c = c + jnp.concatenate(
            [jnp.zeros((Mt, sh), jnp.float32), c[:, : N - sh]], axis=1
        )
        sh *= 2

    cols = []
    for s in range(1, _S + 1):
        cols.append(
            jnp.sum((c < float(s)).astype(jnp.float32), axis=1, keepdims=True)
        )
    pos = jnp.concatenate(cols, axis=1)
    first = pos[:, 0:1]
    first = jnp.where(first == float(N), 0.0, first)
    pos = jnp.where(pos == float(N), first, pos)
    idx = pos.astype(jnp.int32)
    idx_ref[0] = idx
    flat_ref[0] = idx + b * N


def _ball_query(xyz_t, new_xyz):
    B, _, N = xyz_t.shape
    M = new_xyz.shape[1]
    Mt = 8
    grid = (B, M // Mt)
    out_shape = [
        jax.ShapeDtypeStruct((B, M, _S), jnp.int32),
        jax.ShapeDtypeStruct((B, M, _S), jnp.int32),
    ]
    return pl.pallas_call(
        functools.partial(_ball_body, N=N, Mt=Mt),
        grid=grid,
        in_specs=[
            pl.BlockSpec((1, 3, N), lambda b, j: (b, 0, 0)),
            pl.BlockSpec((1, Mt, 3), lambda b, j: (b, j, 0)),
        ],
        out_specs=[
            pl.BlockSpec((1, Mt, _S), lambda b, j: (b, j, 0)),
            pl.BlockSpec((1, Mt, _S), lambda b, j: (b, j, 0)),
        ],
        out_shape=out_shape,
    )(xyz_t, new_xyz)


def _transpose_body(f_ref, o_ref):
    o_ref[0] = f_ref[0].T


def _features_to_rows(features):
    B, C, N = features.shape
    Nt = 512
    return pl.pallas_call(
        _transpose_body,
        grid=(B, N // Nt),
        in_specs=[pl.BlockSpec((1, C, Nt), lambda b, j: (b, 0, j))],
        out_specs=pl.BlockSpec((1, Nt, C), lambda b, j: (b, j, 0)),
        out_shape=jax.ShapeDtypeStruct((B, N, C), jnp.float32),
    )(features)


def _sc_gather_features(ft_rows, fidx):
    V, C = ft_rows.shape
    T = fidx.shape[1]
    W = 128
    mesh = plsc.VectorSubcoreMesh(core_axis_name="c", subcore_axis_name="s")

    @functools.partial(
        pl.kernel,
        out_type=jax.ShapeDtypeStruct((T, C), jnp.float32),
        mesh=mesh,
    )
    def k(ft_hbm, fi_hbm, gf_hbm):
        def body(fi_v, f_o):
            pltpu.sync_copy(ft_hbm.at[fi_v.at[0]], f_o)

        pltpu.emit_pipeline(
            body,
            grid=(T // W,),
            in_specs=[pl.BlockSpec((1, W), lambda i: (0, i))],
            out_specs=[pl.BlockSpec((W, C), lambda i: (i, 0))],
            core_axis_name=("c", "s"),
            dimension_semantics=(pltpu.PARALLEL,),
        )(fi_hbm, gf_hbm)

    return k(ft_rows, fidx)


def _sc_xyz_diff(xyz_flat, nq_flat, lidx, cidx, B, M, N):
    T = lidx.shape[0]
    per_w = T // _NW
    MS = T // B
    n_chunks = per_w // _SC_LANES
    mesh = plsc.VectorSubcoreMesh(core_axis_name="c", subcore_axis_name="s")
    cp = pltpu.CompilerParams()
    if "needs_layout_passes" in pltpu.CompilerParams.__dataclass_fields__:
        cp = dataclasses.replace(cp, needs_layout_passes=False)

    @functools.partial(
        pl.kernel,
        out_type=jax.ShapeDtypeStruct((B * 3 * MS,), jnp.float32),
        mesh=mesh,
        compiler_params=cp,
        scratch_types=[
            pltpu.VMEM((N,), jnp.float32),
            pltpu.VMEM((N,), jnp.float32),
            pltpu.VMEM((N,), jnp.float32),
            pltpu.VMEM((M,), jnp.float32),
            pltpu.VMEM((M,), jnp.float32),
            pltpu.VMEM((M,), jnp.float32),
            pltpu.VMEM((per_w,), jnp.int32),
            pltpu.VMEM((per_w,), jnp.int32),
            pltpu.VMEM((per_w,), jnp.float32),
            pltpu.VMEM((per_w,), jnp.float32),
            pltpu.VMEM((per_w,), jnp.float32),
            pltpu.SemaphoreType.DMA,
        ],
    )
    def k(xyz_hbm, nq_hbm, li_hbm, ci_hbm, o_hbm,
          px_v, py_v, pz_v, qx_v, qy_v, qz_v, li_v, ci_v,
          dx_v, dy_v, dz_v, sem):
        wid = lax.axis_index("s") * _SC_CORES + lax.axis_index("c")
        base = wid * per_w
        b = base // MS
        local = base - b * MS
        pltpu.async_copy(xyz_hbm.at[pl.ds(b * 3 * N, N)], px_v, sem).wait()
        pltpu.async_copy(xyz_hbm.at[pl.ds(b * 3 * N + N, N)], py_v, sem).wait()
        pltpu.async_copy(xyz_hbm.at[pl.ds(b * 3 * N + 2 * N, N)], pz_v, sem).wait()
        pltpu.async_copy(nq_hbm.at[pl.ds(b * 3 * M, M)], qx_v, sem).wait()
        pltpu.async_copy(nq_hbm.at[pl.ds(b * 3 * M + M, M)], qy_v, sem).wait()
        pltpu.async_copy(nq_hbm.at[pl.ds(b * 3 * M + 2 * M, M)], qz_v, sem).wait()
        pltpu.async_copy(li_hbm.at[pl.ds(base, per_w)], li_v, sem).wait()
        pltpu.async_copy(ci_hbm.at[pl.ds(base, per_w)], ci_v, sem).wait()

        @pl.loop(0, n_chunks)
        def _(i):
            sl = pl.ds(i * _SC_LANES, _SC_LANES)
            ii = li_v[sl]
            cc = ci_v[sl]
            dx_v[sl] = plsc.load_gather(px_v, [ii]) - plsc.load_gather(qx_v, [cc])
            dy_v[sl] = plsc.load_gather(py_v, [ii]) - plsc.load_gather(qy_v, [cc])
            dz_v[sl] = plsc.load_gather(pz_v, [ii]) - plsc.load_gather(qz_v, [cc])

        ob = b * 3 * MS + local
        pltpu.async_copy(dx_v, o_hbm.at[pl.ds(ob, per_w)], sem).wait()
        pltpu.async_copy(dy_v, o_hbm.at[pl.ds(ob + MS, per_w)], sem).wait()
        pltpu.async_copy(dz_v, o_hbm.at[pl.ds(ob + 2 * MS, per_w)], sem).wait()

    return k(xyz_flat, nq_flat, lidx, cidx)


def _assemble_body(gf_ref, dxyz_ref, o_ref):
    o_ref[0, 0:3, :] = dxyz_ref[0]
    o_ref[0, 3:, :] = gf_ref[0].T


def _assemble(gf, dxyz, B, M):
    C = gf.shape[-1]
    MS = M * _S
    Lq = 4096
    gf = gf.reshape(B, MS, C)
    return pl.pallas_call(
        _assemble_body,
        grid=(B, MS // Lq),
        in_specs=[
            pl.BlockSpec((1, Lq, C), lambda b, j: (b, j, 0)),
            pl.BlockSpec((1, 3, Lq), lambda b, j: (b, 0, j)),
        ],
        out_specs=pl.BlockSpec((1, 3 + C, Lq), lambda b, j: (b, 0, j)),
        out_shape=jax.ShapeDtypeStruct((B, 3 + C, MS), jnp.float32),
    )(gf, dxyz)


def kernel(xyz, new_xyz, batch_distances, inds, features):
    del batch_distances, inds
    B, N, _ = xyz.shape
    M = new_xyz.shape[1]
    C = features.shape[1]
    T = B * M * _S

    xyz_t = jnp.transpose(xyz, (0, 2, 1))
    nq_t = jnp.transpose(new_xyz, (0, 2, 1))
    idx, flat_idx = _ball_query(xyz_t, new_xyz)

    ft_rows = _features_to_rows(features).reshape(B * N, C)
    fidx = flat_idx.reshape(1, T)
    lidx = idx.reshape(T)
    cidx = jnp.repeat(
        jnp.tile(jnp.arange(M, dtype=jnp.int32), B), _S, total_repeat_length=T
    )

    gf = _sc_gather_features(ft_rows, fidx)
    dxyz = _sc_xyz_diff(
        xyz_t.reshape(B * 3 * N), nq_t.reshape(B * 3 * M), lidx, cidx, B, M, N
    ).reshape(B, 3, M * _S)
    out = _assemble(gf, dxyz, B, M)
    return out.reshape(B, 3 + C, M, _S), idx


# speedup vs baseline: 10.4664x; 2.2775x over previous
import dataclasses
import functools

import jax
import jax.numpy as jnp
from jax import lax
from jax.experimental import pallas as pl
from jax.experimental.pallas import tpu as pltpu
from jax.experimental.pallas import tpu_sc as plsc

_RADIUS2 = 0.1 * 0.1
_S = 32

_SC_CORES = 2
_SC_SUBCORES = 16
_SC_LANES = 16
_NW = _SC_CORES * _SC_SUBCORES


def _ball_body(xyzt_ref, q_ref, idx_ref, flat_ref, *, N, Mt):
    b = pl.program_id(0)
    px = xyzt_ref[0, 0:1, :]
    py = xyzt_ref[0, 1:2, :]
    pz = xyzt_ref[0, 2:3, :]
    q = q_ref[0]
    qx = q[:, 0:1]
    qy = q[:, 1:2]
    qz = q[:, 2:3]
    p2 = (px * px + py * py) + pz * pz
    q2 = (qx * qx + qy * qy) + qz * qz
    cross = jnp.dot(q, xyzt_ref[0], preferred_element_type=jnp.float32)
    dist2 = (q2 + p2) - 2.0 * cross
    maskf = (dist2 < _RADIUS2).astype(jnp.float32)

    c = maskf
    sh = 1
    while sh < N:
        c = c + jnp.concatenate(
            [jnp.zeros((Mt, sh), jnp.float32), c[:, : N - sh]], axis=1
        )
        sh *= 2

    cols = []
    for s in range(1, _S + 1):
        cols.append(
            jnp.sum((c < float(s)).astype(jnp.float32), axis=1, keepdims=True)
        )
    pos = jnp.concatenate(cols, axis=1)
    first = pos[:, 0:1]
    first = jnp.where(first == float(N), 0.0, first)
    pos = jnp.where(pos == float(N), first, pos)
    idx = pos.astype(jnp.int32)
    idx_ref[0] = idx
    flat_ref[0] = idx + b * N


def _ball_query(xyz_t, new_xyz):
    B, _, N = xyz_t.shape
    M = new_xyz.shape[1]
    Mt = 64
    grid = (B, M // Mt)
    out_shape = [
        jax.ShapeDtypeStruct((B, M, _S), jnp.int32),
        jax.ShapeDtypeStruct((B, M, _S), jnp.int32),
    ]
    return pl.pallas_call(
        functools.partial(_ball_body, N=N, Mt=Mt),
        grid=grid,
        in_specs=[
            pl.BlockSpec((1, 3, N), lambda b, j: (b, 0, 0)),
            pl.BlockSpec((1, Mt, 3), lambda b, j: (b, j, 0)),
        ],
        out_specs=[
            pl.BlockSpec((1, Mt, _S), lambda b, j: (b, j, 0)),
            pl.BlockSpec((1, Mt, _S), lambda b, j: (b, j, 0)),
        ],
        out_shape=out_shape,
    )(xyz_t, new_xyz)


def _transpose_body(f_ref, o_ref):
    o_ref[0] = f_ref[0].T


def _features_to_rows(features):
    B, C, N = features.shape
    Nt = 512
    return pl.pallas_call(
        _transpose_body,
        grid=(B, N // Nt),
        in_specs=[pl.BlockSpec((1, C, Nt), lambda b, j: (b, 0, j))],
        out_specs=pl.BlockSpec((1, Nt, C), lambda b, j: (b, j, 0)),
        out_shape=jax.ShapeDtypeStruct((B, N, C), jnp.float32),
    )(features)


def _sc_gather(ft_rows, xyz_rows, fidx):
    V, C = ft_rows.shape
    T = fidx.shape[1]
    W = 128
    mesh = plsc.VectorSubcoreMesh(core_axis_name="c", subcore_axis_name="s")
    cp = pltpu.CompilerParams()
    if "use_tc_tiling_on_sc" in pltpu.CompilerParams.__dataclass_fields__:
        cp = dataclasses.replace(cp, use_tc_tiling_on_sc=True)

    @functools.partial(
        pl.kernel,
        out_type=[
            jax.ShapeDtypeStruct((T, C), jnp.float32),
            jax.ShapeDtypeStruct((T, C), jnp.float32),
        ],
        mesh=mesh,
        compiler_params=cp,
    )
    def k(ft_hbm, xp_hbm, fi_hbm, gf_hbm, gx_hbm):
        def body(fi_v, f_o, x_o):
            pltpu.sync_copy(ft_hbm.at[fi_v.at[0]], f_o)
            pltpu.sync_copy(xp_hbm.at[fi_v.at[0]], x_o)

        pltpu.emit_pipeline(
            body,
            grid=(T // W,),
            in_specs=[pl.BlockSpec((1, W), lambda i: (0, i))],
            out_specs=[
                pl.BlockSpec((W, C), lambda i: (i, 0)),
                pl.BlockSpec((W, C), lambda i: (i, 0)),
            ],
            core_axis_name=("c", "s"),
            dimension_semantics=(pltpu.PARALLEL,),
        )(fi_hbm, gf_hbm, gx_hbm)

    return k(ft_rows, xyz_rows, fidx)


def _assemble_body(gf_ref, gx_ref, nq_ref, r2_ref, o_ref):
    q_rep = jnp.dot(
        r2_ref[...],
        nq_ref[0],
        preferred_element_type=jnp.float32,
        precision=jax.lax.Precision.HIGHEST,
    )
    d = gx_ref[0][:, 0:3] - q_rep
    o_ref[0, 0:3, :] = d.T
    o_ref[0, 3:, :] = gf_ref[0].T


def _assemble(gf, gx, new_xyz, rep, B, M):
    C = gf.shape[-1]
    MS = M * _S
    Lq = 4096
    Mq = Lq // _S
    gf = gf.reshape(B, MS, C)
    gx = gx.reshape(B, MS, gx.shape[-1])
    return pl.pallas_call(
        _assemble_body,
        grid=(B, MS // Lq),
        in_specs=[
            pl.BlockSpec((1, Lq, C), lambda b, j: (b, j, 0)),
            pl.BlockSpec((1, Lq, gx.shape[-1]), lambda b, j: (b, j, 0)),
            pl.BlockSpec((1, Mq, 3), lambda b, j: (b, j, 0)),
            pl.BlockSpec((Lq, Mq), lambda b, j: (0, 0)),
        ],
        out_specs=pl.BlockSpec((1, 3 + C, Lq), lambda b, j: (b, 0, j)),
        out_shape=jax.ShapeDtypeStruct((B, 3 + C, MS), jnp.float32),
    )(gf, gx, new_xyz, rep)


def kernel(xyz, new_xyz, batch_distances, inds, features):
    del batch_distances, inds
    B, N, _ = xyz.shape
    M = new_xyz.shape[1]
    C = features.shape[1]
    T = B * M * _S

    xyz_t = jnp.transpose(xyz, (0, 2, 1))
    idx, flat_idx = _ball_query(xyz_t, new_xyz)

    ft_rows = _features_to_rows(features).reshape(B * N, C)
    pad_n = jnp.zeros((B, N, C - 3), jnp.float32)
    xyz_rows = jnp.concatenate([xyz, pad_n], axis=-1).reshape(B * N, C)
    fidx = flat_idx.reshape(1, T)
    rep = (
        jnp.arange(4096, dtype=jnp.int32)[:, None] // _S
        == jnp.arange(128, dtype=jnp.int32)[None, :]
    ).astype(jnp.float32)

    gf, gx = _sc_gather(ft_rows, xyz_rows, fidx)
    gx8 = gx[:, 0:8]
    out = _assemble(gf, gx8, new_xyz, rep, B, M)
    return out.reshape(B, 3 + C, M, _S), idx


# speedup vs baseline: 10.5660x; 1.0095x over previous
import dataclasses
import functools

import jax
import jax.numpy as jnp
from jax import lax
from jax.experimental import pallas as pl
from jax.experimental.pallas import tpu as pltpu
from jax.experimental.pallas import tpu_sc as plsc

_RADIUS2 = 0.1 * 0.1
_S = 32

_SC_CORES = 2
_SC_SUBCORES = 16
_SC_LANES = 16
_NW = _SC_CORES * _SC_SUBCORES


def _ball_body(xyzt_ref, q_ref, idx_ref, flat_ref, *, N, Mt):
    b = pl.program_id(0)
    px = xyzt_ref[0, 0:1, :]
    py = xyzt_ref[0, 1:2, :]
    pz = xyzt_ref[0, 2:3, :]
    q = q_ref[0]
    qx = q[:, 0:1]
    qy = q[:, 1:2]
    qz = q[:, 2:3]
    p2 = (px * px + py * py) + pz * pz
    q2 = (qx * qx + qy * qy) + qz * qz
    cross = jnp.dot(q, xyzt_ref[0], preferred_element_type=jnp.float32)
    dist2 = (q2 + p2) - 2.0 * cross
    maskf = (dist2 < _RADIUS2).astype(jnp.float32)

    c = maskf
    sh = 1
    while sh < N:
        c = c + jnp.concatenate(
            [jnp.zeros((Mt, sh), jnp.float32), c[:, : N - sh]], axis=1
        )
        sh *= 2

    cols = []
    for s in range(1, _S + 1):
        cols.append(
            jnp.sum((c < float(s)).astype(jnp.float32), axis=1, keepdims=True)
        )
    pos = jnp.concatenate(cols, axis=1)
    first = pos[:, 0:1]
    first = jnp.where(first == float(N), 0.0, first)
    pos = jnp.where(pos == float(N), first, pos)
    idx = pos.astype(jnp.int32)
    idx_ref[0] = idx
    flat_ref[0] = idx + b * N


def _ball_query(xyz_t, new_xyz):
    B, _, N = xyz_t.shape
    M = new_xyz.shape[1]
    Mt = 128
    grid = (B, M // Mt)
    out_shape = [
        jax.ShapeDtypeStruct((B, M, _S), jnp.int32),
        jax.ShapeDtypeStruct((B, M, _S), jnp.int32),
    ]
    return pl.pallas_call(
        functools.partial(_ball_body, N=N, Mt=Mt),
        grid=grid,
        in_specs=[
            pl.BlockSpec((1, 3, N), lambda b, j: (b, 0, 0)),
            pl.BlockSpec((1, Mt, 3), lambda b, j: (b, j, 0)),
        ],
        out_specs=[
            pl.BlockSpec((1, Mt, _S), lambda b, j: (b, j, 0)),
            pl.BlockSpec((1, Mt, _S), lambda b, j: (b, j, 0)),
        ],
        out_shape=out_shape,
    )(xyz_t, new_xyz)


def _transpose_body(f_ref, o_ref):
    o_ref[0] = f_ref[0].T


def _features_to_rows(features):
    B, C, N = features.shape
    Nt = 512
    return pl.pallas_call(
        _transpose_body,
        grid=(B, N // Nt),
        in_specs=[pl.BlockSpec((1, C, Nt), lambda b, j: (b, 0, j))],
        out_specs=pl.BlockSpec((1, Nt, C), lambda b, j: (b, j, 0)),
        out_shape=jax.ShapeDtypeStruct((B, N, C), jnp.float32),
    )(features)


def _sc_gather(ft_rows, xyz_rows, fidx):
    V, C = ft_rows.shape
    T = fidx.shape[1]
    W = 128
    mesh = plsc.VectorSubcoreMesh(core_axis_name="c", subcore_axis_name="s")
    cp = pltpu.CompilerParams()
    if "use_tc_tiling_on_sc" in pltpu.CompilerParams.__dataclass_fields__:
        cp = dataclasses.replace(cp, use_tc_tiling_on_sc=True)

    @functools.partial(
        pl.kernel,
        out_type=[
            jax.ShapeDtypeStruct((T, C), jnp.float32),
            jax.ShapeDtypeStruct((T, C), jnp.float32),
        ],
        mesh=mesh,
        compiler_params=cp,
    )
    def k(ft_hbm, xp_hbm, fi_hbm, gf_hbm, gx_hbm):
        def body(fi_v, f_o, x_o):
            pltpu.sync_copy(ft_hbm.at[fi_v.at[0]], f_o)
            pltpu.sync_copy(xp_hbm.at[fi_v.at[0]], x_o)

        pltpu.emit_pipeline(
            body,
            grid=(T // W,),
            in_specs=[pl.BlockSpec((1, W), lambda i: (0, i))],
            out_specs=[
                pl.BlockSpec((W, C), lambda i: (i, 0)),
                pl.BlockSpec((W, C), lambda i: (i, 0)),
            ],
            core_axis_name=("c", "s"),
            dimension_semantics=(pltpu.PARALLEL,),
        )(fi_hbm, gf_hbm, gx_hbm)

    return k(ft_rows, xyz_rows, fidx)


def _assemble_body(gf_ref, gx_ref, nq_ref, r2_ref, o_ref):
    q_rep = jnp.dot(
        r2_ref[...],
        nq_ref[0],
        preferred_element_type=jnp.float32,
        precision=jax.lax.Precision.HIGHEST,
    )
    d = gx_ref[0][:, 0:3] - q_rep
    o_ref[0, 0:3, :] = d.T
    o_ref[0, 3:, :] = gf_ref[0].T


def _assemble(gf, gx, new_xyz, rep, B, M):
    C = gf.shape[-1]
    MS = M * _S
    Lq = 4096
    Mq = Lq // _S
    gf = gf.reshape(B, MS, C)
    gx = gx.reshape(B, MS, gx.shape[-1])
    return pl.pallas_call(
        _assemble_body,
        grid=(B, MS // Lq),
        in_specs=[
            pl.BlockSpec((1, Lq, C), lambda b, j: (b, j, 0)),
            pl.BlockSpec((1, Lq, gx.shape[-1]), lambda b, j: (b, j, 0)),
            pl.BlockSpec((1, Mq, 3), lambda b, j: (b, j, 0)),
            pl.BlockSpec((Lq, Mq), lambda b, j: (0, 0)),
        ],
        out_specs=pl.BlockSpec((1, 3 + C, Lq), lambda b, j: (b, 0, j)),
        out_shape=jax.ShapeDtypeStruct((B, 3 + C, MS), jnp.float32),
    )(gf, gx, new_xyz, rep)


def kernel(xyz, new_xyz, batch_distances, inds, features):
    del batch_distances, inds
    B, N, _ = xyz.shape
    M = new_xyz.shape[1]
    C = features.shape[1]
    T = B * M * _S

    xyz_t = jnp.transpose(xyz, (0, 2, 1))
    idx, flat_idx = _ball_query(xyz_t, new_xyz)

    ft_rows = _features_to_rows(features).reshape(B * N, C)
    pad_n = jnp.zeros((B, N, C - 3), jnp.float32)
    xyz_rows = jnp.concatenate([xyz, pad_n], axis=-1).reshape(B * N, C)
    fidx = flat_idx.reshape(1, T)
    rep = (
        jnp.arange(4096, dtype=jnp.int32)[:, None] // _S
        == jnp.arange(128, dtype=jnp.int32)[None, :]
    ).astype(jnp.float32)

    gf, gx = _sc_gather(ft_rows, xyz_rows, fidx)
    gx8 = gx[:, 0:8]
    out = _assemble(gf, gx8, new_xyz, rep, B, M)
    return out.reshape(B, 3 + C, M, _S), idx
